# baseline (speedup 1.0000x reference)
import functools

import jax
import jax.numpy as jnp
from jax import lax
from jax.experimental import pallas as pl
from jax.experimental.pallas import tpu as pltpu
from jax.experimental.pallas import tpu_sc as plsc

VOCAB = 1000000
E = 32
B, V, C = 4096, 50, 20

NC, NS = 2, 16
NW = NC * NS
B_PER_W = B // NW
RPB = 8
NBLK = B_PER_W // RPB
IPR = V * C
NSTEP = 63

_mesh = plsc.VectorSubcoreMesh(core_axis_name="c", subcore_axis_name="s")


@functools.partial(
    pl.kernel,
    out_type=jax.ShapeDtypeStruct((B, V, E), jnp.float32),
    mesh=_mesh,
    scratch_types=[
        pltpu.VMEM((C, V, RPB), jnp.int32),
        pltpu.VMEM((C, V, RPB), jnp.int32),
        pltpu.VMEM((IPR + 8,), jnp.int32),
        pltpu.VMEM((IPR + 8,), jnp.int32),
        pltpu.VMEM((IPR + 8,), jnp.int32),
        pltpu.VMEM((IPR + 8,), jnp.int32),
        pltpu.VMEM((IPR + 8,), jnp.int32),
        pltpu.VMEM((IPR + 8, E), jnp.float32),
        pltpu.VMEM((IPR + 8, E), jnp.float32),
        pltpu.VMEM((IPR + 8, E), jnp.float32),
        pltpu.VMEM((V, E), jnp.float32),
        pltpu.VMEM((V, E), jnp.float32),
        pltpu.SemaphoreType.DMA,
        pltpu.SemaphoreType.DMA,
        pltpu.SemaphoreType.DMA,
        pltpu.SemaphoreType.DMA,
        pltpu.SemaphoreType.DMA,
        pltpu.SemaphoreType.DMA,
    ],
    compiler_params=pltpu.CompilerParams(
        use_tc_tiling_on_sc=False, needs_layout_passes=False),
)
def _emb_sum(idx_hbm, table_hbm, out_hbm, idx_a, idx_b,
             flat0, flat1, flat2, ivbuf, icbuf,
             rows0, rows1, rows2, out_a, out_b, ss, sg0, sg1, sg2, so0, so1):
    flats = (flat0, flat1, flat2)
    rows = (rows0, rows1, rows2)
    sg = (sg0, sg1, sg2)
    outv = (out_a, out_b)
    so = (so0, so1)
    wid = lax.axis_index("s") * NC + lax.axis_index("c")
    tile_b0 = wid * B_PER_W

    lanes = lax.iota(jnp.int32, 16)
    iv = lanes * 0
    ic = lanes
    for k in range(NSTEP):
        ivbuf[pl.ds(16 * k, 16)] = iv
        icbuf[pl.ds(16 * k, 16)] = ic
        ic2 = ic + 16
        wrap = ic2 >= C
        ic = jnp.where(wrap, ic2 - C, ic2)
        iv = jnp.minimum(jnp.where(wrap, iv + 1, iv), V - 1)

    def stage_src(blk_b0):
        return idx_hbm.at[pl.ds(0, C), pl.ds(0, V), pl.ds(blk_b0, RPB)]

    pltpu.async_copy(stage_src(tile_b0), idx_a, ss).wait()

    def do_block(blk, cur, nxt):
        blk_b0 = tile_b0 + blk * RPB

        @pl.when(blk + 1 < NBLK)
        def _():
            pltpu.async_copy(stage_src(blk_b0 + RPB), nxt, ss)

        def flatten_row(r, dst):
            ibv = jnp.full((16,), r, jnp.int32)

            def step(k, c2):
                off = pl.multiple_of(16 * k, 16)
                kv = ivbuf[pl.ds(off, 16)]
                kc = icbuf[pl.ds(off, 16)]
                dst[pl.ds(off, 16)] = plsc.load_gather(cur, [kc, kv, ibv])
                return c2

            lax.fori_loop(0, NSTEP, step, 0)

        cps = {}
        for r in range(RPB + 2):
            if r < RPB:
                q = r % 3
                flatten_row(r, flats[q])
                cps[r] = pltpu.async_copy(
                    table_hbm.at[flats[q]], rows[q], sg[q])
            if r >= 2:
                rr = r - 2
                q, p = rr % 3, rr % 2
                cps[rr].wait()
                rv, ov = rows[q], outv[p]
                gb = blk_b0 + rr

                @pl.when((blk > 0) | (rr >= 2))
                def _():
                    pltpu.make_async_copy(ov, out_hbm.at[gb], so[p]).wait()

                def seg_body(v, c2):
                    sbase = v * C
                    acc0 = rv[sbase, pl.ds(0, 16)]
                    acc1 = rv[sbase, pl.ds(16, 16)]
                    for c in range(1, C):
                        acc0 = acc0 + rv[sbase + c, pl.ds(0, 16)]
                        acc1 = acc1 + rv[sbase + c, pl.ds(16, 16)]
                    ov[v, pl.ds(0, 16)] = acc0
                    ov[v, pl.ds(16, 16)] = acc1
                    return c2

                lax.fori_loop(0, V, seg_body, 0)
                pltpu.async_copy(ov, out_hbm.at[gb], so[p])

        @pl.when(blk + 1 < NBLK)
        def _():
            pltpu.make_async_copy(stage_src(blk_b0 + RPB), nxt, ss).wait()

    def pair_body(t, carry):
        do_block(2 * t, idx_a, idx_b)
        do_block(2 * t + 1, idx_b, idx_a)
        return carry

    lax.fori_loop(0, NBLK // 2, pair_body, 0)

    pltpu.make_async_copy(out_a, out_hbm.at[tile_b0], so0).wait()
    pltpu.make_async_copy(out_b, out_hbm.at[tile_b0], so1).wait()


@jax.jit
def kernel(mb_t, mtd, table):
    del mtd
    idx_t = jnp.transpose(mb_t.astype(jnp.int32), (2, 1, 0))
    idx_t = jnp.pad(idx_t, ((0, 0), (0, 6), (0, 0)))
    return _emb_sum(idx_t, table)


# speedup vs baseline: 1.0103x; 1.0103x over previous
import functools

import jax
import jax.numpy as jnp
from jax import lax
from jax.experimental import pallas as pl
from jax.experimental.pallas import tpu as pltpu
from jax.experimental.pallas import tpu_sc as plsc

VOCAB = 1000000
E = 32
B, V, C = 4096, 50, 20

NC, NS = 2, 16
NW = NC * NS
B_PER_W = B // NW
RPB = 8
NBLK = B_PER_W // RPB
IPR = V * C
NSTEP = 63

_mesh = plsc.VectorSubcoreMesh(core_axis_name="c", subcore_axis_name="s")


@functools.partial(
    pl.kernel,
    out_type=jax.ShapeDtypeStruct((B, V, E), jnp.float32),
    mesh=_mesh,
    scratch_types=[
        pltpu.VMEM((C, V, RPB), jnp.int32),
        pltpu.VMEM((C, V, RPB), jnp.int32),
        pltpu.VMEM((IPR + 8,), jnp.int32),
        pltpu.VMEM((IPR + 8,), jnp.int32),
        pltpu.VMEM((IPR + 8,), jnp.int32),
        pltpu.VMEM((IPR + 8,), jnp.int32),
        pltpu.VMEM((IPR + 8,), jnp.int32),
        pltpu.VMEM((IPR + 8, E), jnp.float32),
        pltpu.VMEM((IPR + 8, E), jnp.float32),
        pltpu.VMEM((IPR + 8, E), jnp.float32),
        pltpu.VMEM((V, E), jnp.float32),
        pltpu.VMEM((V, E), jnp.float32),
        pltpu.SemaphoreType.DMA,
        pltpu.SemaphoreType.DMA,
        pltpu.SemaphoreType.DMA,
        pltpu.SemaphoreType.DMA,
        pltpu.SemaphoreType.DMA,
        pltpu.SemaphoreType.DMA,
    ],
    compiler_params=pltpu.CompilerParams(
        use_tc_tiling_on_sc=False, needs_layout_passes=False),
)
def _emb_sum(idx_hbm, table_hbm, out_hbm, idx_a, idx_b,
             flat0, flat1, flat2, ivbuf, icbuf,
             rows0, rows1, rows2, out_a, out_b, ss, sg0, sg1, sg2, so0, so1):
    flats = (flat0, flat1, flat2)
    rows = (rows0, rows1, rows2)
    sg = (sg0, sg1, sg2)
    outv = (out_a, out_b)
    so = (so0, so1)
    wid = lax.axis_index("s") * NC + lax.axis_index("c")
    tile_b0 = wid * B_PER_W

    lanes = lax.iota(jnp.int32, 16)
    iv = lanes * 0
    ic = lanes
    for k in range(NSTEP):
        ivbuf[pl.ds(16 * k, 16)] = iv
        icbuf[pl.ds(16 * k, 16)] = ic
        ic2 = ic + 16
        wrap = ic2 >= C
        ic = jnp.where(wrap, ic2 - C, ic2)
        iv = jnp.minimum(jnp.where(wrap, iv + 1, iv), V - 1)

    def stage_src(blk_b0):
        return idx_hbm.at[pl.ds(0, C), pl.ds(0, V), pl.ds(blk_b0, RPB)]

    pltpu.async_copy(stage_src(tile_b0), idx_a, ss).wait()

    def do_block(blk, cur, nxt):
        blk_b0 = tile_b0 + blk * RPB

        @pl.when(blk + 1 < NBLK)
        def _():
            pltpu.async_copy(stage_src(blk_b0 + RPB), nxt, ss)

        def flatten_row(r, dst):
            ibv = jnp.full((16,), r, jnp.int32)

            def step(k, c2):
                off = pl.multiple_of(16 * k, 16)
                kv = ivbuf[pl.ds(off, 16)]
                kc = icbuf[pl.ds(off, 16)]
                dst[pl.ds(off, 16)] = plsc.load_gather(cur, [kc, kv, ibv])
                return c2

            lax.fori_loop(0, NSTEP, step, 0)

        cps = {}
        for r in range(RPB + 2):
            if r < RPB:
                q = r % 3
                flatten_row(r, flats[q])
                cps[r] = pltpu.async_copy(
                    table_hbm.at[flats[q]], rows[q], sg[q])
            if r >= 2:
                rr = r - 2
                q, p = rr % 3, rr % 2
                cps[rr].wait()
                rv, ov = rows[q], outv[p]
                gb = blk_b0 + rr

                @pl.when((blk > 0) | (rr >= 2))
                def _():
                    pltpu.make_async_copy(ov, out_hbm.at[gb], so[p]).wait()

                def seg_body(v, c2):
                    sbase = v * C
                    acc0 = rv[sbase, pl.ds(0, 16)]
                    acc1 = rv[sbase, pl.ds(16, 16)]
                    for c in range(1, C):
                        acc0 = acc0 + rv[sbase + c, pl.ds(0, 16)]
                        acc1 = acc1 + rv[sbase + c, pl.ds(16, 16)]
                    ov[v, pl.ds(0, 16)] = acc0
                    ov[v, pl.ds(16, 16)] = acc1
                    return c2

                lax.fori_loop(0, V, seg_body, 0)
                pltpu.async_copy(ov, out_hbm.at[gb], so[p])

        @pl.when(blk + 1 < NBLK)
        def _():
            pltpu.make_async_copy(stage_src(blk_b0 + RPB), nxt, ss).wait()

    def pair_body(t, carry):
        do_block(2 * t, idx_a, idx_b)
        do_block(2 * t + 1, idx_b, idx_a)
        return carry

    lax.fori_loop(0, NBLK // 2, pair_body, 0)

    pltpu.make_async_copy(out_a, out_hbm.at[tile_b0], so0).wait()
    pltpu.make_async_copy(out_b, out_hbm.at[tile_b0], so1).wait()


from jax.experimental.layout import Format, Layout


def _impl(mb_t, mtd, table):
    del mtd
    idx_t = jnp.transpose(mb_t.astype(jnp.int32), (2, 1, 0))
    return _emb_sum(idx_t, table)


_kernel_jit = None


def kernel(mb_t, mtd, table):
    global _kernel_jit
    if _kernel_jit is None:
        fmt = Format(
            Layout(major_to_minor=(0, 1, 2), tiling=((8,), (1024,))),
            jax.sharding.SingleDeviceSharding(jax.devices()[0]),
        )
        _kernel_jit = jax.jit(_impl, out_shardings=fmt)
    return _kernel_jit(mb_t, mtd, table)
